# baseline (speedup 1.0000x reference)
import functools

import jax
import jax.numpy as jnp
from jax import lax
from jax.experimental import pallas as pl
from jax.experimental.pallas import tpu as pltpu
from jax.experimental.pallas import tpu_sc as plsc

NC = 2
NS = 16
CH = 128
K = 2
DK = 8
N0 = 64
N1 = 96


def _make_sc_agg(n_pad, d, e_pad):
    nw = NC * NS
    nchunk = N1
    assert (N0 + N1) * NS * CH == e_pad
    rows_per_tile = n_pad // NS
    nslab = rows_per_tile // CH
    mesh = plsc.VectorSubcoreMesh(core_axis_name="c", subcore_axis_name="s")

    @functools.partial(
        pl.kernel,
        mesh=mesh,
        out_type=(
            jax.ShapeDtypeStruct((NC * n_pad, d), jnp.float32),
            jax.ShapeDtypeStruct((NC * n_pad, d), jnp.float32),
        ),
        scratch_types=[
            pltpu.VMEM((2 * K, CH), jnp.int32),
            pltpu.VMEM((nchunk, CH), jnp.int32),
            pltpu.VMEM((CH,), jnp.int32),
            pltpu.VMEM((K * CH, d), jnp.float32),
            pltpu.VMEM_SHARED((n_pad, d), jnp.float32),
            pltpu.SemaphoreType.DMA,
            pltpu.SemaphoreType.DMA,
            pltpu.SemaphoreType.DMA,
        ],
    )
    def sc_agg(src_hbm, dst_hbm, x_hbm, pad_hbm, ones_hbm, agg_out, deg_out,
               src_v, dst_v, pad_v, rows_v, agg_sh, gsem, ssem, isem):
        c = lax.axis_index("c")
        s = lax.axis_index("s")
        wid = c * NS + s
        row0 = s * rows_per_tile
        slab0 = rows_v.at[pl.ds(0, CH)]

        pltpu.sync_copy(pad_hbm, pad_v)

        def fill_zeros():
            pltpu.async_copy(x_hbm.at[pad_v], slab0, gsem).wait()
            hs = [pltpu.async_copy(slab0, agg_sh.at[pl.ds(row0 + k * CH, CH)],
                                   ssem)
                  for k in range(nslab)]
            for h in hs:
                h.wait()

        def copy_out(out_hbm):
            hs = []
            for k in range(nslab):
                buf = rows_v.at[pl.ds((k % K) * CH, CH)]
                if k >= K:
                    hs[k - K].wait()
                pltpu.sync_copy(agg_sh.at[pl.ds(row0 + k * CH, CH)], buf)
                hs.append(pltpu.async_copy(
                    buf, out_hbm.at[pl.ds(c * n_pad + row0 + k * CH, CH)],
                    ssem))
            for h in hs[max(0, nslab - K):]:
                h.wait()

        chunk0 = lax.select(c == 0, s * N0, NS * N0 + s * N1)

        def phase_edges(my_nchunk):
            pltpu.sync_copy(src_hbm.at[pl.ds(chunk0, K)],
                            src_v.at[pl.ds(0, K)])
            plsc.subcore_barrier()

            def group_body(g, slot):
                ph = pltpu.async_copy(
                    src_hbm.at[pl.ds(chunk0 + (g + 1) * K, K)],
                    src_v.at[pl.ds((1 - slot) * K, K)], isem)
                gh = [pltpu.async_copy(x_hbm.at[src_v.at[slot * K + b]],
                                       rows_v.at[pl.ds(b * CH, CH)], gsem)
                      for b in range(K)]
                for h in gh:
                    h.wait()
                sh = [pltpu.async_copy(rows_v.at[pl.ds(b * CH, CH)],
                                       agg_sh.at[dst_v.at[g * K + b]],
                                       ssem, add=True)
                      for b in range(K)]
                for h in sh:
                    h.wait()
                ph.wait()

            def super_group(t, _):
                group_body(2 * t, 0)
                group_body(2 * t + 1, 1)
                return 0

            lax.fori_loop(0, my_nchunk // (2 * K), super_group, 0)

        def phase_deg(my_nchunk):
            def dgroup(g, _):
                hs = [pltpu.async_copy(slab0,
                                       agg_sh.at[dst_v.at[g * DK + b]],
                                       ssem, add=True)
                      for b in range(DK)]
                for h in hs:
                    h.wait()
                return 0

            lax.fori_loop(0, my_nchunk // DK, dgroup, 0)

        pltpu.sync_copy(dst_hbm.at[pl.ds(chunk0, N1)], dst_v)
        fill_zeros()

        @pl.when(c == 0)
        def _():
            phase_edges(N0)

        @pl.when(c == 1)
        def _():
            phase_edges(N1)

        plsc.subcore_barrier()
        copy_out(agg_out)
        plsc.subcore_barrier()

        fill_zeros()
        plsc.subcore_barrier()
        pltpu.sync_copy(ones_hbm, slab0)

        @pl.when(c == 0)
        def _():
            phase_deg(N0)

        @pl.when(c == 1)
        def _():
            phase_deg(N1)

        plsc.subcore_barrier()
        copy_out(deg_out)

    return sc_agg


def _tc_body(x_ref, ap_ref, dp_ref, ws_ref, wn_ref, b_ref, o_ref):
    agg = ap_ref[0] + ap_ref[1]
    deg = dp_ref[0, :, 0:1] + dp_ref[1, :, 0:1]
    mean = agg / jnp.clip(deg, 1.0, None)
    h = lax.dot_general(x_ref[...], ws_ref[...], (((1,), (1,)), ((), ())),
                        preferred_element_type=jnp.float32)
    h = h + lax.dot_general(mean, wn_ref[...], (((1,), (1,)), ((), ())),
                            preferred_element_type=jnp.float32)
    o_ref[...] = jnp.maximum(h + b_ref[...], 0.0)


def kernel(x, edge_index, W_self, W_neigh, b):
    n, d = x.shape
    e = edge_index.shape[1]
    nw = NC * NS
    e_pad = (N0 + N1) * NS * CH
    assert e_pad >= e
    n_pad = ((n + 1 + NS * CH - 1) // (NS * CH)) * (NS * CH)

    src = edge_index[0].astype(jnp.int32)
    dst = edge_index[1].astype(jnp.int32)
    pad_idx = jnp.full((e_pad - e,), n, jnp.int32)
    src_p = jnp.concatenate(
        [src, pad_idx, jnp.full((K * CH,), n, jnp.int32)]).reshape(-1, CH)
    dst_p = jnp.concatenate([dst, pad_idx]).reshape(-1, CH)
    x_pad = jnp.pad(x, ((0, n_pad - n), (0, 0)))
    pad_vec = jnp.full((CH,), n, jnp.int32)
    ones_blk = jnp.ones((CH, d), jnp.float32)

    agg_parts, deg_parts = _make_sc_agg(n_pad, d, e_pad)(
        src_p, dst_p, x_pad, pad_vec, ones_blk)
    agg_parts = agg_parts.reshape(NC, n_pad, d)
    deg_parts = deg_parts.reshape(NC, n_pad, d)

    blk = 1024
    grid = (n_pad // blk,)
    out = pl.pallas_call(
        _tc_body,
        grid=grid,
        in_specs=[
            pl.BlockSpec((blk, d), lambda i: (i, 0)),
            pl.BlockSpec((NC, blk, d), lambda i: (0, i, 0)),
            pl.BlockSpec((NC, blk, d), lambda i: (0, i, 0)),
            pl.BlockSpec((d, d), lambda i: (0, 0)),
            pl.BlockSpec((d, d), lambda i: (0, 0)),
            pl.BlockSpec((1, d), lambda i: (0, 0)),
        ],
        out_specs=pl.BlockSpec((blk, d), lambda i: (i, 0)),
        out_shape=jax.ShapeDtypeStruct((n_pad, d), jnp.float32),
    )(x_pad, agg_parts, deg_parts, W_self, W_neigh, b.reshape(1, d))
    return out[:n]


# speedup vs baseline: 1.0486x; 1.0486x over previous
import functools

import jax
import jax.numpy as jnp
from jax import lax
from jax.experimental import pallas as pl
from jax.experimental.pallas import tpu as pltpu
from jax.experimental.pallas import tpu_sc as plsc

NC = 2
NS = 16
CH = 128
K = 2
DK = 8
N0 = 96
N1 = 64


def _make_sc_agg(n_pad, d, e_pad):
    nw = NC * NS
    nchunk = max(N0, N1)
    assert (N0 + N1) * NS * CH == e_pad
    rows_per_tile = n_pad // NS
    nslab = rows_per_tile // CH
    mesh = plsc.VectorSubcoreMesh(core_axis_name="c", subcore_axis_name="s")

    @functools.partial(
        pl.kernel,
        mesh=mesh,
        out_type=(
            jax.ShapeDtypeStruct((NC * n_pad, d), jnp.float32),
            jax.ShapeDtypeStruct((NC * n_pad, d), jnp.float32),
        ),
        scratch_types=[
            pltpu.VMEM((2 * K, CH), jnp.int32),
            pltpu.VMEM((nchunk, CH), jnp.int32),
            pltpu.VMEM((CH,), jnp.int32),
            pltpu.VMEM((K * CH, d), jnp.float32),
            pltpu.VMEM_SHARED((n_pad, d), jnp.float32),
            pltpu.SemaphoreType.DMA,
            pltpu.SemaphoreType.DMA,
            pltpu.SemaphoreType.DMA,
        ],
    )
    def sc_agg(src_hbm, dst_hbm, x_hbm, pad_hbm, ones_hbm, agg_out, deg_out,
               src_v, dst_v, pad_v, rows_v, agg_sh, gsem, ssem, isem):
        c = lax.axis_index("c")
        s = lax.axis_index("s")
        wid = c * NS + s
        row0 = s * rows_per_tile
        slab0 = rows_v.at[pl.ds(0, CH)]

        pltpu.sync_copy(pad_hbm, pad_v)

        def fill_zeros():
            pltpu.async_copy(x_hbm.at[pad_v], slab0, gsem).wait()
            hs = [pltpu.async_copy(slab0, agg_sh.at[pl.ds(row0 + k * CH, CH)],
                                   ssem)
                  for k in range(nslab)]
            for h in hs:
                h.wait()

        def copy_out(out_hbm):
            hs = []
            for k in range(nslab):
                buf = rows_v.at[pl.ds((k % K) * CH, CH)]
                if k >= K:
                    hs[k - K].wait()
                pltpu.sync_copy(agg_sh.at[pl.ds(row0 + k * CH, CH)], buf)
                hs.append(pltpu.async_copy(
                    buf, out_hbm.at[pl.ds(c * n_pad + row0 + k * CH, CH)],
                    ssem))
            for h in hs[max(0, nslab - K):]:
                h.wait()

        chunk0 = lax.select(c == 0, s * N0, NS * N0 + s * N1)

        def phase_edges(my_nchunk):
            pltpu.sync_copy(src_hbm.at[pl.ds(chunk0, K)],
                            src_v.at[pl.ds(0, K)])
            plsc.subcore_barrier()

            def group_body(g, slot):
                ph = pltpu.async_copy(
                    src_hbm.at[pl.ds(chunk0 + (g + 1) * K, K)],
                    src_v.at[pl.ds((1 - slot) * K, K)], isem)
                gh = [pltpu.async_copy(x_hbm.at[src_v.at[slot * K + b]],
                                       rows_v.at[pl.ds(b * CH, CH)], gsem)
                      for b in range(K)]
                for h in gh:
                    h.wait()
                sh = [pltpu.async_copy(rows_v.at[pl.ds(b * CH, CH)],
                                       agg_sh.at[dst_v.at[g * K + b]],
                                       ssem, add=True)
                      for b in range(K)]
                for h in sh:
                    h.wait()
                ph.wait()

            def super_group(t, _):
                group_body(2 * t, 0)
                group_body(2 * t + 1, 1)
                return 0

            lax.fori_loop(0, my_nchunk // (2 * K), super_group, 0)

        def phase_deg(my_nchunk):
            def dgroup(g, _):
                hs = [pltpu.async_copy(slab0,
                                       agg_sh.at[dst_v.at[g * DK + b]],
                                       ssem, add=True)
                      for b in range(DK)]
                for h in hs:
                    h.wait()
                return 0

            lax.fori_loop(0, my_nchunk // DK, dgroup, 0)

        pltpu.sync_copy(dst_hbm.at[pl.ds(chunk0, max(N0, N1))], dst_v)
        fill_zeros()

        @pl.when(c == 0)
        def _():
            phase_edges(N0)

        @pl.when(c == 1)
        def _():
            phase_edges(N1)

        plsc.subcore_barrier()
        copy_out(agg_out)
        plsc.subcore_barrier()

        fill_zeros()
        plsc.subcore_barrier()
        pltpu.sync_copy(ones_hbm, slab0)

        @pl.when(c == 0)
        def _():
            phase_deg(N0)

        @pl.when(c == 1)
        def _():
            phase_deg(N1)

        plsc.subcore_barrier()
        copy_out(deg_out)

    return sc_agg


def _tc_body(x_ref, ap_ref, dp_ref, ws_ref, wn_ref, b_ref, o_ref):
    agg = ap_ref[0] + ap_ref[1]
    deg = dp_ref[0, :, 0:1] + dp_ref[1, :, 0:1]
    mean = agg / jnp.clip(deg, 1.0, None)
    h = lax.dot_general(x_ref[...], ws_ref[...], (((1,), (1,)), ((), ())),
                        preferred_element_type=jnp.float32)
    h = h + lax.dot_general(mean, wn_ref[...], (((1,), (1,)), ((), ())),
                            preferred_element_type=jnp.float32)
    o_ref[...] = jnp.maximum(h + b_ref[...], 0.0)


def kernel(x, edge_index, W_self, W_neigh, b):
    n, d = x.shape
    e = edge_index.shape[1]
    nw = NC * NS
    e_pad = (N0 + N1) * NS * CH
    assert e_pad >= e
    n_pad = ((n + 1 + NS * CH - 1) // (NS * CH)) * (NS * CH)

    src = edge_index[0].astype(jnp.int32)
    dst = edge_index[1].astype(jnp.int32)
    pad_idx = jnp.full((e_pad - e,), n, jnp.int32)
    src_p = jnp.concatenate(
        [src, pad_idx, jnp.full((K * CH,), n, jnp.int32)]).reshape(-1, CH)
    dst_p = jnp.concatenate(
        [dst, pad_idx,
         jnp.full((abs(N0 - N1) * NS * CH,), n, jnp.int32)]).reshape(-1, CH)
    x_pad = jnp.pad(x, ((0, n_pad - n), (0, 0)))
    pad_vec = jnp.full((CH,), n, jnp.int32)
    ones_blk = jnp.ones((CH, d), jnp.float32)

    agg_parts, deg_parts = _make_sc_agg(n_pad, d, e_pad)(
        src_p, dst_p, x_pad, pad_vec, ones_blk)
    agg_parts = agg_parts.reshape(NC, n_pad, d)
    deg_parts = deg_parts.reshape(NC, n_pad, d)

    blk = 1024
    grid = (n_pad // blk,)
    out = pl.pallas_call(
        _tc_body,
        grid=grid,
        in_specs=[
            pl.BlockSpec((blk, d), lambda i: (i, 0)),
            pl.BlockSpec((NC, blk, d), lambda i: (0, i, 0)),
            pl.BlockSpec((NC, blk, d), lambda i: (0, i, 0)),
            pl.BlockSpec((d, d), lambda i: (0, 0)),
            pl.BlockSpec((d, d), lambda i: (0, 0)),
            pl.BlockSpec((1, d), lambda i: (0, 0)),
        ],
        out_specs=pl.BlockSpec((blk, d), lambda i: (i, 0)),
        out_shape=jax.ShapeDtypeStruct((n_pad, d), jnp.float32),
    )(x_pad, agg_parts, deg_parts, W_self, W_neigh, b.reshape(1, d))
    return out[:n]
